# baseline (speedup 1.0000x reference)
import functools

import jax
import jax.numpy as jnp
from jax import lax
from jax.experimental import pallas as pl
from jax.experimental.pallas import tpu as pltpu
from jax.experimental.pallas import tpu_sc as plsc

H = 512
B = 16384

NC, NS, L = 2, 16, 16
NW = NC * NS
RPW = B // NW
C = 64
NCHUNK = RPW // C
NG = C // L
UNROLL = 4


def _sc_diff(user_table, item_table, user, item, neg):
    mesh = plsc.VectorSubcoreMesh(core_axis_name="c", subcore_axis_name="s")

    @functools.partial(
        pl.kernel,
        mesh=mesh,
        out_type=jax.ShapeDtypeStruct((B,), jnp.float32),
        compiler_params=pltpu.CompilerParams(use_tc_tiling_on_sc=False,
                                             needs_layout_passes=False),
        scratch_types=[
            pltpu.VMEM((RPW,), jnp.int32),
            pltpu.VMEM((RPW,), jnp.int32),
            pltpu.VMEM((RPW,), jnp.int32),
            pltpu.VMEM((C, H), jnp.float32),
            pltpu.VMEM((C, H), jnp.float32),
            pltpu.VMEM((C, H), jnp.float32),
            pltpu.VMEM((RPW,), jnp.float32),
            pltpu.SemaphoreType.DMA,
        ],
    )
    def k(ut_hbm, it_hbm, u_hbm, i_hbm, n_hbm, out_hbm,
          uidx, iidx, nidx, ubuf, ibuf, nbuf, outv, sem):
        wid = lax.axis_index("s") * NC + lax.axis_index("c")
        base = wid * RPW
        pltpu.sync_copy(u_hbm.at[pl.ds(base, RPW)], uidx)
        pltpu.sync_copy(i_hbm.at[pl.ds(base, RPW)], iidx)
        pltpu.sync_copy(n_hbm.at[pl.ds(base, RPW)], nidx)

        lane = lax.iota(jnp.int32, L)

        for c in range(NCHUNK):
            cu = pltpu.async_copy(ut_hbm.at[uidx.at[pl.ds(c * C, C)]], ubuf, sem)
            ci = pltpu.async_copy(it_hbm.at[iidx.at[pl.ds(c * C, C)]], ibuf, sem)
            cn = pltpu.async_copy(it_hbm.at[nidx.at[pl.ds(c * C, C)]], nbuf, sem)
            cu.wait()
            ci.wait()
            cn.wait()
            for g in range(NG):
                rows = lane + (g * L)

                def step(t, acc):
                    h0 = t * UNROLL
                    for kk in range(UNROLL):
                        col = jnp.full((L,), 0, jnp.int32) + (h0 + kk)
                        uu = plsc.load_gather(ubuf, [rows, col])
                        ii = plsc.load_gather(ibuf, [rows, col])
                        nn = plsc.load_gather(nbuf, [rows, col])
                        acc = acc + uu * (ii - nn)
                    return acc

                acc = lax.fori_loop(0, H // UNROLL, step,
                                    jnp.zeros((L,), jnp.float32))
                outv[pl.ds(c * C + g * L, L)] = acc

        pltpu.sync_copy(outv, out_hbm.at[pl.ds(base, RPW)])

    return k(user_table, item_table, user, item, neg)


def _tc_loss(diff2d):
    def body(d_ref, o_ref):
        x = d_ref[...]
        nls = jnp.log(1.0 + jnp.exp(-jnp.abs(x))) - jnp.minimum(x, 0.0)
        o_ref[0, 0] = jnp.sum(nls)

    out = pl.pallas_call(
        body,
        out_shape=jax.ShapeDtypeStruct((1, 1), jnp.float32),
        out_specs=pl.BlockSpec(memory_space=pltpu.SMEM),
    )(diff2d)
    return out[0, 0]


def kernel(user_table, item_table, user, item, neg):
    diff = _sc_diff(user_table, item_table,
                    user.astype(jnp.int32), item.astype(jnp.int32),
                    neg.astype(jnp.int32))
    return _tc_loss(diff.reshape(B // 128, 128))


# speedup vs baseline: 2.0399x; 2.0399x over previous
import functools

import jax
import jax.numpy as jnp
from jax import lax
from jax.experimental import pallas as pl
from jax.experimental.pallas import tpu as pltpu
from jax.experimental.pallas import tpu_sc as plsc

H = 512
B = 16384

NC, NS, L = 2, 16, 16
NW = NC * NS
RPW = B // NW
C = 32
NCHUNK = RPW // C
NG = C // L
HC = H // L


def _sc_diff(user_table, item_table, user, item, neg):
    mesh = plsc.VectorSubcoreMesh(core_axis_name="c", subcore_axis_name="s")

    @functools.partial(
        pl.kernel,
        mesh=mesh,
        out_type=jax.ShapeDtypeStruct((B,), jnp.float32),
        compiler_params=pltpu.CompilerParams(use_tc_tiling_on_sc=False,
                                             needs_layout_passes=False),
        scratch_types=[
            pltpu.VMEM((RPW,), jnp.int32),
            pltpu.VMEM((RPW,), jnp.int32),
            pltpu.VMEM((RPW,), jnp.int32),
            pltpu.VMEM((C, H), jnp.float32),
            pltpu.VMEM((C, H), jnp.float32),
            pltpu.VMEM((C, H), jnp.float32),
            pltpu.VMEM((C, H), jnp.float32),
            pltpu.VMEM((C, H), jnp.float32),
            pltpu.VMEM((C, H), jnp.float32),
            pltpu.VMEM((RPW,), jnp.float32),
            pltpu.SemaphoreType.DMA,
            pltpu.SemaphoreType.DMA,
        ],
    )
    def k(ut_hbm, it_hbm, u_hbm, i_hbm, n_hbm, out_hbm,
          uidx, iidx, nidx, ub0, ib0, nb0, ub1, ib1, nb1, outv, s0, s1):
        wid = lax.axis_index("s") * NC + lax.axis_index("c")
        base = wid * RPW
        pltpu.sync_copy(u_hbm.at[pl.ds(base, RPW)], uidx)
        pltpu.sync_copy(i_hbm.at[pl.ds(base, RPW)], iidx)
        pltpu.sync_copy(n_hbm.at[pl.ds(base, RPW)], nidx)

        bufs = ((ub0, ib0, nb0, s0), (ub1, ib1, nb1, s1))
        lane = lax.iota(jnp.int32, L)

        def issue(c, par):
            ub, ib, nb, sem = bufs[par]
            off = pl.multiple_of(c * C, C)
            pltpu.async_copy(ut_hbm.at[uidx.at[pl.ds(off, C)]], ub, sem)
            pltpu.async_copy(it_hbm.at[iidx.at[pl.ds(off, C)]], ib, sem)
            pltpu.async_copy(it_hbm.at[nidx.at[pl.ds(off, C)]], nb, sem)

        def drain(par):
            ub, ib, nb, sem = bufs[par]
            pltpu.make_async_copy(ut_hbm.at[pl.ds(0, C)], ub, sem).wait()
            pltpu.make_async_copy(it_hbm.at[pl.ds(0, C)], ib, sem).wait()
            pltpu.make_async_copy(it_hbm.at[pl.ds(0, C)], nb, sem).wait()

        issue(0, 0)

        def chunk_body(cc, carry):
            for par in range(2):
                c = cc + par
                ub, ib, nb, _sem = bufs[par]

                @pl.when(c + 1 < NCHUNK)
                def _():
                    issue(c + 1, 1 - par)

                drain(par)

                for g in range(NG):
                    def step(t, accs):
                        hs = pl.multiple_of(t * L, L)
                        out = []
                        for r in range(L):
                            row = g * L + r
                            uu = ub[row, pl.ds(hs, L)]
                            ii = ib[row, pl.ds(hs, L)]
                            nn = nb[row, pl.ds(hs, L)]
                            out.append(accs[r] + uu * (ii - nn))
                        return tuple(out)

                    accs = lax.fori_loop(
                        0, HC, step,
                        tuple(jnp.zeros((L,), jnp.float32) for _ in range(L)))
                    res = jnp.zeros((L,), jnp.float32)
                    for r in range(L):
                        res = jnp.where(lane == r, jnp.sum(accs[r]), res)
                    outv[pl.ds(c * C + g * L, L)] = res
            return carry

        lax.fori_loop(0, NCHUNK // 2, lambda i, x: chunk_body(i * 2, x), 0)

        pltpu.sync_copy(outv, out_hbm.at[pl.ds(base, RPW)])

    return k(user_table, item_table, user, item, neg)


def _tc_loss(diff2d):
    def body(d_ref, o_ref):
        x = d_ref[...]
        nls = jnp.log(1.0 + jnp.exp(-jnp.abs(x))) - jnp.minimum(x, 0.0)
        o_ref[0, 0] = jnp.sum(nls)

    out = pl.pallas_call(
        body,
        out_shape=jax.ShapeDtypeStruct((1, 1), jnp.float32),
        out_specs=pl.BlockSpec(memory_space=pltpu.SMEM),
    )(diff2d)
    return out[0, 0]


def kernel(user_table, item_table, user, item, neg):
    diff = _sc_diff(user_table, item_table,
                    user.astype(jnp.int32), item.astype(jnp.int32),
                    neg.astype(jnp.int32))
    return _tc_loss(diff.reshape(B // 128, 128))


# speedup vs baseline: 10.9763x; 5.3808x over previous
import functools

import jax
import jax.numpy as jnp
from jax import lax
from jax.experimental import pallas as pl
from jax.experimental.pallas import tpu as pltpu
from jax.experimental.pallas import tpu_sc as plsc

H = 512
B = 16384

NC, NS, L = 2, 16, 16
NW = NC * NS
RPW = B // NW
C = 32
NCHUNK = RPW // C
NG = C // L
HC = H // L


def _sc_diff(user_table, item_table, user, item, neg):
    mesh = plsc.VectorSubcoreMesh(core_axis_name="c", subcore_axis_name="s")

    @functools.partial(
        pl.kernel,
        mesh=mesh,
        out_type=jax.ShapeDtypeStruct((B,), jnp.float32),
        compiler_params=pltpu.CompilerParams(use_tc_tiling_on_sc=True,
                                             needs_layout_passes=False),
        scratch_types=[
            pltpu.VMEM((RPW,), jnp.int32),
            pltpu.VMEM((RPW,), jnp.int32),
            pltpu.VMEM((RPW,), jnp.int32),
            pltpu.VMEM((C, H), jnp.float32),
            pltpu.VMEM((C, H), jnp.float32),
            pltpu.VMEM((C, H), jnp.float32),
            pltpu.VMEM((C, H), jnp.float32),
            pltpu.VMEM((C, H), jnp.float32),
            pltpu.VMEM((C, H), jnp.float32),
            pltpu.VMEM((RPW,), jnp.float32),
            pltpu.SemaphoreType.DMA,
            pltpu.SemaphoreType.DMA,
        ],
    )
    def k(ut_hbm, it_hbm, u_hbm, i_hbm, n_hbm, out_hbm,
          uidx, iidx, nidx, ub0, ib0, nb0, ub1, ib1, nb1, outv, s0, s1):
        wid = lax.axis_index("s") * NC + lax.axis_index("c")
        base = wid * RPW
        pltpu.sync_copy(u_hbm.at[pl.ds(base, RPW)], uidx)
        pltpu.sync_copy(i_hbm.at[pl.ds(base, RPW)], iidx)
        pltpu.sync_copy(n_hbm.at[pl.ds(base, RPW)], nidx)

        bufs = ((ub0, ib0, nb0, s0), (ub1, ib1, nb1, s1))
        lane = lax.iota(jnp.int32, L)

        def issue(c, par):
            ub, ib, nb, sem = bufs[par]
            off = pl.multiple_of(c * C, C)
            pltpu.async_copy(ut_hbm.at[uidx.at[pl.ds(off, C)]], ub, sem)
            pltpu.async_copy(it_hbm.at[iidx.at[pl.ds(off, C)]], ib, sem)
            pltpu.async_copy(it_hbm.at[nidx.at[pl.ds(off, C)]], nb, sem)

        def drain(par):
            ub, ib, nb, sem = bufs[par]
            pltpu.make_async_copy(ut_hbm.at[pl.ds(0, C)], ub, sem).wait()
            pltpu.make_async_copy(it_hbm.at[pl.ds(0, C)], ib, sem).wait()
            pltpu.make_async_copy(it_hbm.at[pl.ds(0, C)], nb, sem).wait()

        issue(0, 0)

        def chunk_body(cc, carry):
            for par in range(2):
                c = cc + par
                ub, ib, nb, _sem = bufs[par]

                @pl.when(c + 1 < NCHUNK)
                def _():
                    issue(c + 1, 1 - par)

                drain(par)

                for g in range(NG):
                    def step(t, accs):
                        hs = pl.multiple_of(t * L, L)
                        out = []
                        for r in range(L):
                            row = g * L + r
                            uu = ub[row, pl.ds(hs, L)]
                            ii = ib[row, pl.ds(hs, L)]
                            nn = nb[row, pl.ds(hs, L)]
                            out.append(accs[r] + uu * (ii - nn))
                        return tuple(out)

                    accs = lax.fori_loop(
                        0, HC, step,
                        tuple(jnp.zeros((L,), jnp.float32) for _ in range(L)))
                    res = jnp.zeros((L,), jnp.float32)
                    for r in range(L):
                        res = jnp.where(lane == r, jnp.sum(accs[r]), res)
                    outv[pl.ds(c * C + g * L, L)] = res
            return carry

        lax.fori_loop(0, NCHUNK // 2, lambda i, x: chunk_body(i * 2, x), 0)

        pltpu.sync_copy(outv, out_hbm.at[pl.ds(base, RPW)])

    return k(user_table, item_table, user, item, neg)


def _tc_loss(diff2d):
    def body(d_ref, o_ref):
        x = d_ref[...]
        nls = jnp.log(1.0 + jnp.exp(-jnp.abs(x))) - jnp.minimum(x, 0.0)
        o_ref[0, 0] = jnp.sum(nls)

    out = pl.pallas_call(
        body,
        out_shape=jax.ShapeDtypeStruct((1, 1), jnp.float32),
        out_specs=pl.BlockSpec(memory_space=pltpu.SMEM),
    )(diff2d)
    return out[0, 0]


def kernel(user_table, item_table, user, item, neg):
    diff = _sc_diff(user_table, item_table,
                    user.astype(jnp.int32), item.astype(jnp.int32),
                    neg.astype(jnp.int32))
    return _tc_loss(diff.reshape(B // 128, 128))
